# Initial kernel scaffold; baseline (speedup 1.0000x reference)
import jax
import jax.numpy as jnp
from jax.experimental import pallas as pl


def kernel(task_state_scheduled, task_state_ready, task_length, task_completion_time, task_memory_req_mb, task_cpu_req_cores, vm_completion_time, vm_speed, vm_energy_rate, vm_memory_mb, vm_available_memory_mb, vm_used_memory_fraction, vm_active_tasks_count, vm_cpu_cores, vm_available_cpu_cores, vm_used_cpu_fraction_cores, compatibilities, task_dependencies, params):
    raise NotImplementedError("write your pallas kernel here")


# speedup vs baseline: 6.4542x; 6.4542x over previous
import functools

import jax
import jax.numpy as jnp
from jax import lax
from jax.experimental import pallas as pl
from jax.experimental.pallas import tpu as pltpu
from jax.experimental.pallas import tpu_sc as plsc

NT = 50000
NV = 50000
N = NT + NV
E_TV = 1200000
E_DEP = 400000
E = E_TV + E_DEP
HID = 32
EMB = 16

NC = 2
NS = 16
CHUNK = 640
SUB = 128
NSUB = CHUNK // SUB
TV_CHUNKS = E_TV // CHUNK
DEP_CHUNKS = E_DEP // CHUNK
ZR = 1000
NZCHUNKS = N // ZR

_MESH = dict(core_axis_name="c", subcore_axis_name="s", num_cores=NC,
             num_subcores=NS)


def _sc_conv_body(split_by_core, table_halves, tvs, tvd, dps, dpd, table, agg,
                  sidx, didx, rows, zbuf, acc, sem):
  c = lax.axis_index("c")
  s = lax.axis_index("s")

  def _zb(i, carry):
    zbuf[i, :] = jnp.zeros((EMB,), jnp.float32)
    return carry
  lax.fori_loop(0, ZR, _zb, 0)
  for r in range(-(-NZCHUNKS // NS)):
    k = s + r * NS

    @pl.when(k < NZCHUNKS)
    def _():
      pltpu.sync_copy(zbuf, acc.at[pl.ds(k * ZR, ZR)])
  plsc.subcore_barrier()

  if split_by_core:
    wid = c * NS + s
    nwork = NC * NS
  else:
    wid = s
    nwork = NS

  def _process(esrc, edst, total_chunks, dst_off):
    count = total_chunks // nwork + jnp.where(wid < total_chunks % nwork, 1, 0)

    def chunk_body(i, carry):
      k = wid + i * nwork
      base = pl.multiple_of(k * CHUNK, CHUNK)
      pltpu.sync_copy(esrc.at[pl.ds(base, CHUNK)], sidx)
      pltpu.sync_copy(edst.at[pl.ds(base, CHUNK)], didx)
      descs = []
      for j in range(NSUB):
        ids = sidx.at[pl.ds(j * SUB, SUB)]
        if table_halves:
          src = table.at[c].at[ids]
        else:
          src = table.at[ids]
        descs.append(pltpu.async_copy(src, rows.at[pl.ds(j * SUB, SUB)], sem))
      for d in descs:
        d.wait()
      descs = []
      for j in range(NSUB):
        idd = didx.at[pl.ds(j * SUB, SUB)]
        if dst_off:
          tgt = acc.at[pl.ds(dst_off, N - dst_off)].at[idd]
        else:
          tgt = acc.at[idd]
        descs.append(
            pltpu.async_copy(rows.at[pl.ds(j * SUB, SUB)], tgt, sem, add=True))
      for d in descs:
        d.wait()
      return carry

    lax.fori_loop(0, count, chunk_body, 0)

  _process(tvs, tvd, TV_CHUNKS, NT)
  _process(dps, dpd, DEP_CHUNKS, 0)
  plsc.subcore_barrier()
  for r in range(-(-NZCHUNKS // NS)):
    k = s + r * NS

    @pl.when(k < NZCHUNKS)
    def _():
      off = pl.multiple_of(k * ZR, ZR)
      pltpu.sync_copy(acc.at[pl.ds(off, ZR)], agg.at[c, pl.ds(off, ZR)])


def _make_sc_conv(split_by_core):
  body = functools.partial(_sc_conv_body, split_by_core, not split_by_core)
  return pl.kernel(
      body,
      out_type=jax.ShapeDtypeStruct((NC, N, EMB), jnp.float32),
      mesh=plsc.VectorSubcoreMesh(**_MESH),
      compiler_params=pltpu.CompilerParams(use_tc_tiling_on_sc=False),
      scratch_types=[
          pltpu.VMEM((CHUNK,), jnp.int32),
          pltpu.VMEM((CHUNK,), jnp.int32),
          pltpu.VMEM((CHUNK, EMB), jnp.float32),
          pltpu.VMEM((ZR, EMB), jnp.float32),
          pltpu.VMEM_SHARED((N, EMB), jnp.float32),
          pltpu.SemaphoreType.DMA,
      ],
  )


CR = CHUNK // 8


def _sc_edge_body(tvs, tvd, dps, dpd, emb, outs, outd, sidx, didx, rows_s,
                  rows_d, r128, sem):
  c = lax.axis_index("c")
  s = lax.axis_index("s")
  wid = c * NS + s
  nwork = NC * NS

  def _process(esrc, edst, total_chunks, dst_off, out_base):
    count = total_chunks // nwork + jnp.where(wid < total_chunks % nwork, 1, 0)

    def chunk_body(i, carry):
      k = wid + i * nwork
      base = pl.multiple_of(k * CHUNK, CHUNK)
      pltpu.sync_copy(esrc.at[pl.ds(base, CHUNK)], sidx)
      pltpu.sync_copy(edst.at[pl.ds(base, CHUNK)], didx)
      descs = []
      for j in range(NSUB):
        ids = sidx.at[pl.ds(j * SUB, SUB)]
        descs.append(pltpu.async_copy(
            emb.at[ids], rows_s.at[pl.ds(j * SUB, SUB)], sem))
        idd = didx.at[pl.ds(j * SUB, SUB)]
        if dst_off:
          src_d = emb.at[pl.ds(dst_off, N - dst_off)].at[idd]
        else:
          src_d = emb.at[idd]
        descs.append(pltpu.async_copy(
            src_d, rows_d.at[pl.ds(j * SUB, SUB)], sem))
      for d in descs:
        d.wait()
      ob = (out_base + base) // 8
      r128[...] = rows_s[...].reshape(CR, 128)
      pltpu.sync_copy(r128, outs.at[pl.ds(ob, CR)])
      r128[...] = rows_d[...].reshape(CR, 128)
      pltpu.sync_copy(r128, outd.at[pl.ds(ob, CR)])
      return carry

    lax.fori_loop(0, count, chunk_body, 0)

  _process(tvs, tvd, TV_CHUNKS, NT, 0)
  _process(dps, dpd, DEP_CHUNKS, 0, E_TV)


_sc_edge = pl.kernel(
    _sc_edge_body,
    out_type=[jax.ShapeDtypeStruct((E // 8, 128), jnp.float32),
              jax.ShapeDtypeStruct((E // 8, 128), jnp.float32)],
    mesh=plsc.VectorSubcoreMesh(**_MESH),
    compiler_params=pltpu.CompilerParams(use_tc_tiling_on_sc=False),
    scratch_types=[
        pltpu.VMEM((CHUNK,), jnp.int32),
        pltpu.VMEM((CHUNK,), jnp.int32),
        pltpu.VMEM((CHUNK, EMB), jnp.float32),
        pltpu.VMEM((CHUNK, EMB), jnp.float32),
        pltpu.VMEM((CR, 128), jnp.float32),
        pltpu.SemaphoreType.DMA,
    ],
)


def _maxc_body(v_ref, o_ref):
  o_ref[...] = jnp.broadcast_to(jnp.maximum(jnp.max(v_ref[...]), 1.0), (1, 1))


def _max_cores(vcc):
  return pl.pallas_call(
      _maxc_body,
      out_shape=jax.ShapeDtypeStruct((1, 1), jnp.float32),
  )(vcc)


def _enc1_body(is_task, nrows, x_ref, mc_ref, w_ref, b_ref,
               y_ref, s_ref, ss_ref):
  i = pl.program_id(0)
  x = x_ref[...]
  mc = mc_ref[0, 0]
  cols = lax.broadcasted_iota(jnp.int32, x.shape, 1)
  if is_task:
    x = jnp.where(cols == 4, x / 1000.0, x)
    x = jnp.where(cols == 5, x / mc, x)
  else:
    x = jnp.where(cols == 1, 1.0 / (x + 1e-8), x)
    x = jnp.where((cols == 3) | (cols == 4), x / 1000.0, x)
    x = jnp.where((cols == 7) | (cols == 8), x / mc, x)
  y = jnp.dot(x, w_ref[...], preferred_element_type=jnp.float32) + b_ref[...]
  y_ref[...] = y
  ps = jnp.sum(y, axis=0, keepdims=True)
  pss = jnp.sum(y * y, axis=0, keepdims=True)

  @pl.when(i == 0)
  def _():
    s_ref[...] = ps
    ss_ref[...] = pss

  @pl.when(i > 0)
  def _():
    s_ref[...] += ps
    ss_ref[...] += pss


def _bn_from_stats(y, s_ref, ss_ref, g_ref, be_ref, nrows):
  m = s_ref[...] / nrows
  v = ss_ref[...] / nrows - m * m
  return (y - m) / jnp.sqrt(v + 1e-5) * g_ref[...] + be_ref[...]


def _enc2_body(nrows, y_ref, s_ref, ss_ref, g_ref, be_ref, w_ref, b_ref,
               y2_ref, s2_ref, ss2_ref):
  i = pl.program_id(0)
  h = jnp.maximum(_bn_from_stats(y_ref[...], s_ref, ss_ref, g_ref, be_ref,
                                 nrows), 0.0)
  y2 = jnp.dot(h, w_ref[...], preferred_element_type=jnp.float32) + b_ref[...]
  y2_ref[...] = y2
  ps = jnp.sum(y2, axis=0, keepdims=True)
  pss = jnp.sum(y2 * y2, axis=0, keepdims=True)

  @pl.when(i == 0)
  def _():
    s2_ref[...] = ps
    ss2_ref[...] = pss

  @pl.when(i > 0)
  def _():
    s2_ref[...] += ps
    ss2_ref[...] += pss


def _enc3_body(nrows, y_ref, s_ref, ss_ref, g_ref, be_ref, w_ref, b_ref,
               o_ref):
  h = jnp.maximum(_bn_from_stats(y_ref[...], s_ref, ss_ref, g_ref, be_ref,
                                 nrows), 0.0)
  o_ref[...] = (jnp.dot(h, w_ref[...], preferred_element_type=jnp.float32)
                + b_ref[...])


def _full(shape):
  return pl.BlockSpec(shape, lambda i: (0,) * len(shape))


def _encoder(x_raw, mc, p, pre, d_in):
  nrows = x_raw.shape[0]
  B = 2000
  grid = nrows // B
  row_spec = lambda w: pl.BlockSpec((B, w), lambda i: (i, 0))
  stat_shape = lambda w: jax.ShapeDtypeStruct((1, w), jnp.float32)

  y1, s1, ss1 = pl.pallas_call(
      functools.partial(_enc1_body, pre == "te", float(nrows)),
      grid=(grid,),
      in_specs=[row_spec(d_in), _full((1, 1)), _full((d_in, HID)),
                _full((1, HID))],
      out_specs=[row_spec(HID), _full((1, HID)), _full((1, HID))],
      out_shape=[jax.ShapeDtypeStruct((nrows, HID), jnp.float32),
                 stat_shape(HID), stat_shape(HID)],
  )(x_raw, mc, p[pre + "_w1"], p[pre + "_b1"].reshape(1, HID))

  y2, s2, ss2 = pl.pallas_call(
      functools.partial(_enc2_body, float(nrows)),
      grid=(grid,),
      in_specs=[row_spec(HID)] + [_full((1, HID))] * 4 +
               [_full((HID, HID)), _full((1, HID))],
      out_specs=[row_spec(HID), _full((1, HID)), _full((1, HID))],
      out_shape=[jax.ShapeDtypeStruct((nrows, HID), jnp.float32),
                 stat_shape(HID), stat_shape(HID)],
  )(y1, s1, ss1, p[pre + "_g1"].reshape(1, HID), p[pre + "_be1"].reshape(1, HID),
    p[pre + "_w2"], p[pre + "_b2"].reshape(1, HID))

  return pl.pallas_call(
      functools.partial(_enc3_body, float(nrows)),
      grid=(grid,),
      in_specs=[row_spec(HID)] + [_full((1, HID))] * 4 +
               [_full((HID, EMB)), _full((1, EMB))],
      out_specs=row_spec(EMB),
      out_shape=jax.ShapeDtypeStruct((nrows, EMB), jnp.float32),
  )(y2, s2, ss2, p[pre + "_g2"].reshape(1, HID), p[pre + "_be2"].reshape(1, HID),
    p[pre + "_w3"], p[pre + "_b3"].reshape(1, EMB))


def _conv_a_body(x_halves, x_ref, agg_ref, w_ref, b_ref,
                 y_ref, s_ref, ss_ref):
  i = pl.program_id(0)
  if x_halves:
    x = jnp.concatenate([x_ref[0], x_ref[1]], axis=1)
    a = jnp.concatenate([agg_ref[0], agg_ref[1]], axis=1)
  else:
    x = x_ref[...]
    a = agg_ref[0] + agg_ref[1]
  t = x + a
  y = jnp.dot(t, w_ref[...], preferred_element_type=jnp.float32) + b_ref[...]
  y_ref[...] = y
  ps = jnp.sum(y, axis=0, keepdims=True)
  pss = jnp.sum(y * y, axis=0, keepdims=True)

  @pl.when(i == 0)
  def _():
    s_ref[...] = ps
    ss_ref[...] = pss

  @pl.when(i > 0)
  def _():
    s_ref[...] += ps
    ss_ref[...] += pss


def _conv_b_body(final_relu, ngrid, y_ref, s_ref, ss_ref, g_ref, be_ref,
                 w_ref, b_ref, h_ref):
  h = jnp.maximum(_bn_from_stats(y_ref[...], s_ref, ss_ref, g_ref, be_ref,
                                 float(N)), 0.0)
  z = jnp.dot(h, w_ref[...], preferred_element_type=jnp.float32) + b_ref[...]
  if final_relu:
    z = jnp.maximum(z, 0.0)
  h_ref[0] = z[:, :EMB]
  h_ref[1] = z[:, EMB:]


def _conv_b3_body(ngrid, y_ref, s_ref, ss_ref, g_ref, be_ref, w_ref, b_ref,
                  h_ref, gs_ref):
  i = pl.program_id(0)
  h = jnp.maximum(_bn_from_stats(y_ref[...], s_ref, ss_ref, g_ref, be_ref,
                                 float(N)), 0.0)
  z = jnp.dot(h, w_ref[...], preferred_element_type=jnp.float32) + b_ref[...]
  h_ref[...] = z
  ps = jnp.sum(z, axis=0, keepdims=True)

  @pl.when(i == 0)
  def _():
    gs_ref[...] = ps

  @pl.when(i > 0)
  def _():
    gs_ref[...] += ps

  @pl.when(i == ngrid - 1)
  def _():
    gs_ref[...] = gs_ref[...] / float(N)


def _conv_mlp(x, agg, p, pre, d_in, d_hid, d_out, final_relu, is_last):
  B = 2000
  grid = N // B
  x_halves = x.ndim == 3
  row_spec = lambda w: pl.BlockSpec((B, w), lambda i: (i, 0))
  half_spec = pl.BlockSpec((NC, B, EMB), lambda i: (0, i, 0))
  x_spec = half_spec if x_halves else row_spec(EMB)
  stat_shape = jax.ShapeDtypeStruct((1, d_hid), jnp.float32)

  y, s, ss = pl.pallas_call(
      functools.partial(_conv_a_body, x_halves),
      grid=(grid,),
      in_specs=[x_spec, half_spec, _full((d_in, d_hid)),
                _full((1, d_hid))],
      out_specs=[row_spec(d_hid), _full((1, d_hid)), _full((1, d_hid))],
      out_shape=[jax.ShapeDtypeStruct((N, d_hid), jnp.float32),
                 stat_shape, stat_shape],
  )(x, agg, p[pre + "_w1"], p[pre + "_b1"].reshape(1, d_hid))

  common_in = [row_spec(d_hid)] + [_full((1, d_hid))] * 4 + \
              [_full((d_hid, d_out)), _full((1, d_out))]
  args = (y, s, ss, p[pre + "_g1"].reshape(1, d_hid),
          p[pre + "_be1"].reshape(1, d_hid), p[pre + "_w2"],
          p[pre + "_b2"].reshape(1, d_out))
  if not is_last:
    return pl.pallas_call(
        functools.partial(_conv_b_body, final_relu, grid),
        grid=(grid,),
        in_specs=common_in,
        out_specs=half_spec,
        out_shape=jax.ShapeDtypeStruct((NC, N, EMB), jnp.float32),
    )(*args)
  return pl.pallas_call(
      functools.partial(_conv_b3_body, grid),
      grid=(grid,),
      in_specs=common_in,
      out_specs=[row_spec(d_out), _full((1, d_out))],
      out_shape=[jax.ShapeDtypeStruct((N, d_out), jnp.float32),
                 jax.ShapeDtypeStruct((1, d_out), jnp.float32)],
  )(*args)


@jax.jit
def _forward(tss, tsr, tl, tct, tmr, tcr, vct, vs, ver, vmm, vam, vumf,
             vatc, vcc, vacc, vucf, compatibilities, task_dependencies,
             params):
  mc = _max_cores(vcc)
  task_x = jnp.stack([tss, tsr, tl, tct, tmr, tcr], axis=-1)
  vm_x = jnp.stack([vct, vs, ver, vmm, vam, vumf, vatc, vcc, vacc, vucf],
                   axis=-1)
  task_h = _encoder(task_x, mc, params, "te", 6)
  vm_h = _encoder(vm_x, mc, params, "ve", 10)
  node_x = jnp.concatenate([task_h, vm_h], axis=0)

  tvs = compatibilities[0].astype(jnp.int32)
  tvd = compatibilities[1].astype(jnp.int32)
  dps = task_dependencies[0].astype(jnp.int32)
  dpd = task_dependencies[1].astype(jnp.int32)

  agg1 = _make_sc_conv(True)(tvs, tvd, dps, dpd, node_x)
  h1 = _conv_mlp(node_x, agg1, params, "g1", EMB, HID, HID, True, False)
  agg2 = _make_sc_conv(False)(tvs, tvd, dps, dpd, h1)
  h2 = _conv_mlp(h1, agg2, params, "g2", HID, HID, HID, True, False)
  agg3 = _make_sc_conv(False)(tvs, tvd, dps, dpd, h2)
  node_emb, graph_emb = _conv_mlp(h2, agg3, params, "g3", HID, EMB, EMB,
                                  False, True)
  eouts, eoutd = _sc_edge(tvs, tvd, dps, dpd, node_emb)
  edge_emb = jnp.concatenate([eouts.reshape(E, EMB), eoutd.reshape(E, EMB)],
                             axis=1)
  return node_emb, edge_emb, graph_emb


def kernel(task_state_scheduled, task_state_ready, task_length,
           task_completion_time, task_memory_req_mb, task_cpu_req_cores,
           vm_completion_time, vm_speed, vm_energy_rate, vm_memory_mb,
           vm_available_memory_mb, vm_used_memory_fraction,
           vm_active_tasks_count, vm_cpu_cores, vm_available_cpu_cores,
           vm_used_cpu_fraction_cores, compatibilities, task_dependencies,
           params):
  return _forward(task_state_scheduled, task_state_ready, task_length,
                  task_completion_time, task_memory_req_mb,
                  task_cpu_req_cores, vm_completion_time, vm_speed,
                  vm_energy_rate, vm_memory_mb, vm_available_memory_mb,
                  vm_used_memory_fraction, vm_active_tasks_count,
                  vm_cpu_cores, vm_available_cpu_cores,
                  vm_used_cpu_fraction_cores, compatibilities,
                  task_dependencies, params)


# speedup vs baseline: 8.7536x; 1.3563x over previous
import functools

import jax
import jax.numpy as jnp
from jax import lax
from jax.experimental import pallas as pl
from jax.experimental.pallas import tpu as pltpu
from jax.experimental.pallas import tpu_sc as plsc

NT = 50000
NV = 50000
N = NT + NV
E_TV = 1200000
E_DEP = 400000
E = E_TV + E_DEP
HID = 32
EMB = 16

NC = 2
NS = 16
CHUNK = 640
SUB = 128
NSUB = CHUNK // SUB
TV_CHUNKS = E_TV // CHUNK
DEP_CHUNKS = E_DEP // CHUNK
ZR = 1000
NZCHUNKS = N // ZR

_MESH = dict(core_axis_name="c", subcore_axis_name="s", num_cores=NC,
             num_subcores=NS)


def _sc_conv_body(split_by_core, table_halves, tvs, tvd, dps, dpd, table, agg,
                  sidx, didx, rows, zbuf, acc, sem):
  c = lax.axis_index("c")
  s = lax.axis_index("s")

  def _zb(i, carry):
    zbuf[i, :] = jnp.zeros((EMB,), jnp.float32)
    return carry
  lax.fori_loop(0, ZR, _zb, 0)
  for r in range(-(-NZCHUNKS // NS)):
    k = s + r * NS

    @pl.when(k < NZCHUNKS)
    def _():
      pltpu.sync_copy(zbuf, acc.at[pl.ds(k * ZR, ZR)])
  plsc.subcore_barrier()

  if split_by_core:
    wid = c * NS + s
    nwork = NC * NS
  else:
    wid = s
    nwork = NS

  def _process(esrc, edst, total_chunks, dst_off):
    count = total_chunks // nwork + jnp.where(wid < total_chunks % nwork, 1, 0)

    def chunk_body(i, carry):
      k = wid + i * nwork
      base = pl.multiple_of(k * CHUNK, CHUNK)
      pltpu.sync_copy(esrc.at[pl.ds(base, CHUNK)], sidx)
      pltpu.sync_copy(edst.at[pl.ds(base, CHUNK)], didx)
      descs = []
      for j in range(NSUB):
        ids = sidx.at[pl.ds(j * SUB, SUB)]
        if table_halves:
          src = table.at[c].at[ids]
        else:
          src = table.at[ids]
        descs.append(pltpu.async_copy(src, rows.at[pl.ds(j * SUB, SUB)], sem))
      for d in descs:
        d.wait()
      descs = []
      for j in range(NSUB):
        idd = didx.at[pl.ds(j * SUB, SUB)]
        if dst_off:
          tgt = acc.at[pl.ds(dst_off, N - dst_off)].at[idd]
        else:
          tgt = acc.at[idd]
        descs.append(
            pltpu.async_copy(rows.at[pl.ds(j * SUB, SUB)], tgt, sem, add=True))
      for d in descs:
        d.wait()
      return carry

    lax.fori_loop(0, count, chunk_body, 0)

  _process(tvs, tvd, TV_CHUNKS, NT)
  _process(dps, dpd, DEP_CHUNKS, 0)
  plsc.subcore_barrier()
  for r in range(-(-NZCHUNKS // NS)):
    k = s + r * NS

    @pl.when(k < NZCHUNKS)
    def _():
      off = pl.multiple_of(k * ZR, ZR)
      pltpu.sync_copy(acc.at[pl.ds(off, ZR)], agg.at[c, pl.ds(off, ZR)])


def _make_sc_conv(split_by_core):
  body = functools.partial(_sc_conv_body, split_by_core, not split_by_core)
  return pl.kernel(
      body,
      out_type=jax.ShapeDtypeStruct((NC, N, EMB), jnp.float32),
      mesh=plsc.VectorSubcoreMesh(**_MESH),
      compiler_params=pltpu.CompilerParams(use_tc_tiling_on_sc=False),
      scratch_types=[
          pltpu.VMEM((CHUNK,), jnp.int32),
          pltpu.VMEM((CHUNK,), jnp.int32),
          pltpu.VMEM((CHUNK, EMB), jnp.float32),
          pltpu.VMEM((ZR, EMB), jnp.float32),
          pltpu.VMEM_SHARED((N, EMB), jnp.float32),
          pltpu.SemaphoreType.DMA,
      ],
  )


ZR2 = 250
NZ2 = NT // ZR2


def _sc_conv32_body(tvs, tvd, dps, dpd, table, agg, sidx, didx, rows, zbuf,
                    acc, sem):
  c = lax.axis_index("c")
  s = lax.axis_index("s")

  def _zb(i, carry):
    zbuf[i, :] = jnp.zeros((HID,), jnp.float32)
    return carry
  lax.fori_loop(0, ZR2, _zb, 0)
  for r in range(-(-NZ2 // NS)):
    k = s + r * NS

    @pl.when(k < NZ2)
    def _():
      pltpu.sync_copy(zbuf, acc.at[pl.ds(k * ZR2, ZR2)])
  plsc.subcore_barrier()

  def _process(esrc, edst, total_chunks):
    count = total_chunks // NS + jnp.where(s < total_chunks % NS, 1, 0)

    def chunk_body(i, carry):
      k = s + i * NS
      base = pl.multiple_of(k * CHUNK, CHUNK)
      pltpu.sync_copy(esrc.at[pl.ds(base, CHUNK)], sidx)
      pltpu.sync_copy(edst.at[pl.ds(base, CHUNK)], didx)
      descs = []
      for j in range(NSUB):
        ids = sidx.at[pl.ds(j * SUB, SUB)]
        descs.append(pltpu.async_copy(table.at[ids],
                                      rows.at[pl.ds(j * SUB, SUB)], sem))
      for d in descs:
        d.wait()
      descs = []
      for j in range(NSUB):
        idd = didx.at[pl.ds(j * SUB, SUB)]
        descs.append(
            pltpu.async_copy(rows.at[pl.ds(j * SUB, SUB)], acc.at[idd], sem,
                             add=True))
      for d in descs:
        d.wait()
      return carry

    lax.fori_loop(0, count, chunk_body, 0)

  @pl.when(c == 1)
  def _():
    _process(tvs, tvd, TV_CHUNKS)

  @pl.when(c == 0)
  def _():
    _process(dps, dpd, DEP_CHUNKS)

  plsc.subcore_barrier()
  for r in range(-(-NZ2 // NS)):
    k = s + r * NS

    @pl.when(k < NZ2)
    def _():
      off = pl.multiple_of(k * ZR2, ZR2)
      pltpu.sync_copy(acc.at[pl.ds(off, ZR2)], agg.at[c, pl.ds(off, ZR2)])


_sc_conv32 = pl.kernel(
    _sc_conv32_body,
    out_type=jax.ShapeDtypeStruct((NC, NT, HID), jnp.float32),
    mesh=plsc.VectorSubcoreMesh(**_MESH),
    compiler_params=pltpu.CompilerParams(use_tc_tiling_on_sc=False),
    scratch_types=[
        pltpu.VMEM((CHUNK,), jnp.int32),
        pltpu.VMEM((CHUNK,), jnp.int32),
        pltpu.VMEM((CHUNK, HID), jnp.float32),
        pltpu.VMEM((ZR2, HID), jnp.float32),
        pltpu.VMEM_SHARED((NT, HID), jnp.float32),
        pltpu.SemaphoreType.DMA,
    ],
)


def _sc_edge_body(tvs, tvd, dps, dpd, emb, out, sidx, didx, rows_s, rows_d,
                  sem):
  c = lax.axis_index("c")
  s = lax.axis_index("s")
  wid = c * NS + s
  nwork = NC * NS

  def _process(esrc, edst, total_chunks, dst_off, out_base):
    count = total_chunks // nwork + jnp.where(wid < total_chunks % nwork, 1, 0)

    def chunk_body(i, carry):
      k = wid + i * nwork
      base = pl.multiple_of(k * CHUNK, CHUNK)
      pltpu.sync_copy(esrc.at[pl.ds(base, CHUNK)], sidx)
      pltpu.sync_copy(edst.at[pl.ds(base, CHUNK)], didx)
      descs = []
      for j in range(NSUB):
        ids = sidx.at[pl.ds(j * SUB, SUB)]
        descs.append(pltpu.async_copy(
            emb.at[ids], rows_s.at[pl.ds(j * SUB, SUB)], sem))
        idd = didx.at[pl.ds(j * SUB, SUB)]
        if dst_off:
          src_d = emb.at[pl.ds(dst_off, N - dst_off)].at[idd]
        else:
          src_d = emb.at[idd]
        descs.append(pltpu.async_copy(
            src_d, rows_d.at[pl.ds(j * SUB, SUB)], sem))
      for d in descs:
        d.wait()
      ob = out_base + base
      pltpu.sync_copy(rows_s, out.at[pl.ds(ob, CHUNK), pl.ds(0, EMB)])
      pltpu.sync_copy(rows_d, out.at[pl.ds(ob, CHUNK), pl.ds(EMB, EMB)])
      return carry

    lax.fori_loop(0, count, chunk_body, 0)

  _process(tvs, tvd, TV_CHUNKS, NT, 0)
  _process(dps, dpd, DEP_CHUNKS, 0, E_TV)


_sc_edge = pl.kernel(
    _sc_edge_body,
    out_type=jax.ShapeDtypeStruct((E, 2 * EMB), jnp.float32),
    mesh=plsc.VectorSubcoreMesh(**_MESH),
    compiler_params=pltpu.CompilerParams(use_tc_tiling_on_sc=False),
    scratch_types=[
        pltpu.VMEM((CHUNK,), jnp.int32),
        pltpu.VMEM((CHUNK,), jnp.int32),
        pltpu.VMEM((CHUNK, EMB), jnp.float32),
        pltpu.VMEM((CHUNK, EMB), jnp.float32),
        pltpu.SemaphoreType.DMA,
    ],
)


def _maxc_body(v_ref, o_ref):
  o_ref[...] = jnp.broadcast_to(jnp.maximum(jnp.max(v_ref[...]), 1.0), (1, 1))


def _max_cores(vcc):
  return pl.pallas_call(
      _maxc_body,
      out_shape=jax.ShapeDtypeStruct((1, 1), jnp.float32),
  )(vcc)


def _enc1_body(is_task, nrows, x_ref, mc_ref, w_ref, b_ref,
               y_ref, s_ref, ss_ref):
  i = pl.program_id(0)
  x = x_ref[...]
  mc = mc_ref[0, 0]
  cols = lax.broadcasted_iota(jnp.int32, x.shape, 1)
  if is_task:
    x = jnp.where(cols == 4, x / 1000.0, x)
    x = jnp.where(cols == 5, x / mc, x)
  else:
    x = jnp.where(cols == 1, 1.0 / (x + 1e-8), x)
    x = jnp.where((cols == 3) | (cols == 4), x / 1000.0, x)
    x = jnp.where((cols == 7) | (cols == 8), x / mc, x)
  y = jnp.dot(x, w_ref[...], preferred_element_type=jnp.float32) + b_ref[...]
  y_ref[...] = y
  ps = jnp.sum(y, axis=0, keepdims=True)
  pss = jnp.sum(y * y, axis=0, keepdims=True)

  @pl.when(i == 0)
  def _():
    s_ref[...] = ps
    ss_ref[...] = pss

  @pl.when(i > 0)
  def _():
    s_ref[...] += ps
    ss_ref[...] += pss


def _bn_from_stats(y, s_ref, ss_ref, g_ref, be_ref, nrows):
  m = s_ref[...] / nrows
  v = ss_ref[...] / nrows - m * m
  return (y - m) / jnp.sqrt(v + 1e-5) * g_ref[...] + be_ref[...]


def _enc2_body(nrows, y_ref, s_ref, ss_ref, g_ref, be_ref, w_ref, b_ref,
               y2_ref, s2_ref, ss2_ref):
  i = pl.program_id(0)
  h = jnp.maximum(_bn_from_stats(y_ref[...], s_ref, ss_ref, g_ref, be_ref,
                                 nrows), 0.0)
  y2 = jnp.dot(h, w_ref[...], preferred_element_type=jnp.float32) + b_ref[...]
  y2_ref[...] = y2
  ps = jnp.sum(y2, axis=0, keepdims=True)
  pss = jnp.sum(y2 * y2, axis=0, keepdims=True)

  @pl.when(i == 0)
  def _():
    s2_ref[...] = ps
    ss2_ref[...] = pss

  @pl.when(i > 0)
  def _():
    s2_ref[...] += ps
    ss2_ref[...] += pss


def _enc3_body(nrows, y_ref, s_ref, ss_ref, g_ref, be_ref, w_ref, b_ref,
               o_ref):
  h = jnp.maximum(_bn_from_stats(y_ref[...], s_ref, ss_ref, g_ref, be_ref,
                                 nrows), 0.0)
  o_ref[...] = (jnp.dot(h, w_ref[...], preferred_element_type=jnp.float32)
                + b_ref[...])


def _full(shape):
  return pl.BlockSpec(shape, lambda i: (0,) * len(shape))


def _encoder(x_raw, mc, p, pre, d_in):
  nrows = x_raw.shape[0]
  B = 2000
  grid = nrows // B
  row_spec = lambda w: pl.BlockSpec((B, w), lambda i: (i, 0))
  stat_shape = lambda w: jax.ShapeDtypeStruct((1, w), jnp.float32)

  y1, s1, ss1 = pl.pallas_call(
      functools.partial(_enc1_body, pre == "te", float(nrows)),
      grid=(grid,),
      in_specs=[row_spec(d_in), _full((1, 1)), _full((d_in, HID)),
                _full((1, HID))],
      out_specs=[row_spec(HID), _full((1, HID)), _full((1, HID))],
      out_shape=[jax.ShapeDtypeStruct((nrows, HID), jnp.float32),
                 stat_shape(HID), stat_shape(HID)],
  )(x_raw, mc, p[pre + "_w1"], p[pre + "_b1"].reshape(1, HID))

  y2, s2, ss2 = pl.pallas_call(
      functools.partial(_enc2_body, float(nrows)),
      grid=(grid,),
      in_specs=[row_spec(HID)] + [_full((1, HID))] * 4 +
               [_full((HID, HID)), _full((1, HID))],
      out_specs=[row_spec(HID), _full((1, HID)), _full((1, HID))],
      out_shape=[jax.ShapeDtypeStruct((nrows, HID), jnp.float32),
                 stat_shape(HID), stat_shape(HID)],
  )(y1, s1, ss1, p[pre + "_g1"].reshape(1, HID), p[pre + "_be1"].reshape(1, HID),
    p[pre + "_w2"], p[pre + "_b2"].reshape(1, HID))

  return pl.pallas_call(
      functools.partial(_enc3_body, float(nrows)),
      grid=(grid,),
      in_specs=[row_spec(HID)] + [_full((1, HID))] * 4 +
               [_full((HID, EMB)), _full((1, EMB))],
      out_specs=row_spec(EMB),
      out_shape=jax.ShapeDtypeStruct((nrows, EMB), jnp.float32),
  )(y2, s2, ss2, p[pre + "_g2"].reshape(1, HID), p[pre + "_be2"].reshape(1, HID),
    p[pre + "_w3"], p[pre + "_b3"].reshape(1, EMB))


def _conv_a_body(partial2, x_ref, agg_ref, w_ref, b_ref,
                 y_ref, s_ref, ss_ref):
  i = pl.program_id(0)
  x = x_ref[...]
  if partial2:
    a = agg_ref[0] + agg_ref[1]
  else:
    a = agg_ref[...]
  t = x + a
  y = jnp.dot(t, w_ref[...], preferred_element_type=jnp.float32) + b_ref[...]
  y_ref[...] = y
  ps = jnp.sum(y, axis=0, keepdims=True)
  pss = jnp.sum(y * y, axis=0, keepdims=True)

  @pl.when(i == 0)
  def _():
    s_ref[...] = ps
    ss_ref[...] = pss

  @pl.when(i > 0)
  def _():
    s_ref[...] += ps
    ss_ref[...] += pss


def _conv_b_body(final_relu, ngrid, y_ref, s_ref, ss_ref, g_ref, be_ref,
                 w_ref, b_ref, h_ref):
  h = jnp.maximum(_bn_from_stats(y_ref[...], s_ref, ss_ref, g_ref, be_ref,
                                 float(N)), 0.0)
  z = jnp.dot(h, w_ref[...], preferred_element_type=jnp.float32) + b_ref[...]
  if final_relu:
    z = jnp.maximum(z, 0.0)
  h_ref[...] = z


def _conv_b3_body(ngrid, y_ref, s_ref, ss_ref, g_ref, be_ref, w_ref, b_ref,
                  h_ref, gs_ref):
  i = pl.program_id(0)
  h = jnp.maximum(_bn_from_stats(y_ref[...], s_ref, ss_ref, g_ref, be_ref,
                                 float(N)), 0.0)
  z = jnp.dot(h, w_ref[...], preferred_element_type=jnp.float32) + b_ref[...]
  h_ref[...] = z
  ps = jnp.sum(z, axis=0, keepdims=True)

  @pl.when(i == 0)
  def _():
    gs_ref[...] = ps

  @pl.when(i > 0)
  def _():
    gs_ref[...] += ps

  @pl.when(i == ngrid - 1)
  def _():
    gs_ref[...] = gs_ref[...] / float(N)


def _conv_mlp(x, agg, p, pre, d_in, d_hid, d_out, final_relu, is_last):
  B = 2000
  grid = N // B
  partial2 = agg.ndim == 3
  row_spec = lambda w: pl.BlockSpec((B, w), lambda i: (i, 0))
  half_spec = pl.BlockSpec((NC, B, EMB), lambda i: (0, i, 0))
  agg_spec = half_spec if partial2 else row_spec(d_in)
  stat_shape = jax.ShapeDtypeStruct((1, d_hid), jnp.float32)

  y, s, ss = pl.pallas_call(
      functools.partial(_conv_a_body, partial2),
      grid=(grid,),
      in_specs=[row_spec(d_in), agg_spec, _full((d_in, d_hid)),
                _full((1, d_hid))],
      out_specs=[row_spec(d_hid), _full((1, d_hid)), _full((1, d_hid))],
      out_shape=[jax.ShapeDtypeStruct((N, d_hid), jnp.float32),
                 stat_shape, stat_shape],
  )(x, agg, p[pre + "_w1"], p[pre + "_b1"].reshape(1, d_hid))

  common_in = [row_spec(d_hid)] + [_full((1, d_hid))] * 4 + \
              [_full((d_hid, d_out)), _full((1, d_out))]
  args = (y, s, ss, p[pre + "_g1"].reshape(1, d_hid),
          p[pre + "_be1"].reshape(1, d_hid), p[pre + "_w2"],
          p[pre + "_b2"].reshape(1, d_out))
  if not is_last:
    return pl.pallas_call(
        functools.partial(_conv_b_body, final_relu, grid),
        grid=(grid,),
        in_specs=common_in,
        out_specs=row_spec(d_out),
        out_shape=jax.ShapeDtypeStruct((N, d_out), jnp.float32),
    )(*args)
  return pl.pallas_call(
      functools.partial(_conv_b3_body, grid),
      grid=(grid,),
      in_specs=common_in,
      out_specs=[row_spec(d_out), _full((1, d_out))],
      out_shape=[jax.ShapeDtypeStruct((N, d_out), jnp.float32),
                 jax.ShapeDtypeStruct((1, d_out), jnp.float32)],
  )(*args)


@jax.jit
def _forward(tss, tsr, tl, tct, tmr, tcr, vct, vs, ver, vmm, vam, vumf,
             vatc, vcc, vacc, vucf, compatibilities, task_dependencies,
             params):
  mc = _max_cores(vcc)
  task_x = jnp.stack([tss, tsr, tl, tct, tmr, tcr], axis=-1)
  vm_x = jnp.stack([vct, vs, ver, vmm, vam, vumf, vatc, vcc, vacc, vucf],
                   axis=-1)
  task_h = _encoder(task_x, mc, params, "te", 6)
  vm_h = _encoder(vm_x, mc, params, "ve", 10)
  node_x = jnp.concatenate([task_h, vm_h], axis=0)

  tvs = compatibilities[0].astype(jnp.int32)
  tvd = compatibilities[1].astype(jnp.int32)
  dps = task_dependencies[0].astype(jnp.int32)
  dpd = task_dependencies[1].astype(jnp.int32)

  agg1 = _make_sc_conv(True)(tvs, tvd, dps, dpd, node_x)
  h1 = _conv_mlp(node_x, agg1, params, "g1", EMB, HID, HID, True, False)
  agg2 = _sc_conv32(tvs, tvd, dps, dpd, h1).reshape(N, HID)
  h2 = _conv_mlp(h1, agg2, params, "g2", HID, HID, HID, True, False)
  agg3 = _sc_conv32(tvs, tvd, dps, dpd, h2).reshape(N, HID)
  node_emb, graph_emb = _conv_mlp(h2, agg3, params, "g3", HID, EMB, EMB,
                                  False, True)
  edge_emb = _sc_edge(tvs, tvd, dps, dpd, node_emb)
  return node_emb, edge_emb, graph_emb


def kernel(task_state_scheduled, task_state_ready, task_length,
           task_completion_time, task_memory_req_mb, task_cpu_req_cores,
           vm_completion_time, vm_speed, vm_energy_rate, vm_memory_mb,
           vm_available_memory_mb, vm_used_memory_fraction,
           vm_active_tasks_count, vm_cpu_cores, vm_available_cpu_cores,
           vm_used_cpu_fraction_cores, compatibilities, task_dependencies,
           params):
  return _forward(task_state_scheduled, task_state_ready, task_length,
                  task_completion_time, task_memory_req_mb,
                  task_cpu_req_cores, vm_completion_time, vm_speed,
                  vm_energy_rate, vm_memory_mb, vm_available_memory_mb,
                  vm_used_memory_fraction, vm_active_tasks_count,
                  vm_cpu_cores, vm_available_cpu_cores,
                  vm_used_cpu_fraction_cores, compatibilities,
                  task_dependencies, params)
